# baseline (speedup 1.0000x reference)
import functools

import jax
import jax.numpy as jnp
from jax import lax
from jax.experimental import pallas as pl
from jax.experimental.pallas import tpu as pltpu
from jax.experimental.pallas import tpu_sc as plsc

N = 10000
E = 320000
D = 128
H = 128
C = 40
CP = 48

NC = 2
NS = 16
L = 16
NW = NC * NS
EPW = E // NW
NPAD = 10240
RPT = NPAD // NS
DGR = NPAD // D

_mesh = plsc.VectorSubcoreMesh(core_axis_name="c", subcore_axis_name="s")


def _seg_sum_body(width, chunk, nb, with_deg, x_hbm, ei_hbm, *rest):
    nchunk = EPW // chunk
    if with_deg:
        out_hbm, deg_hbm = rest[:2]
        rest = rest[2:]
    else:
        out_hbm = rest[0]
        deg_hbm = None
        rest = rest[1:]
    src_v, dst_v = rest[:2]
    rows = list(rest[2:2 + nb])
    gsems = list(rest[2 + nb:2 + 2 * nb])
    rest = rest[2 + 2 * nb:]
    if with_deg:
        deg_v, degidx_v, acc_sh, deg_sh = rest[:4]
    else:
        deg_v = degidx_v = deg_sh = None
        acc_sh = rest[0]

    cid = lax.axis_index("c")
    sid = lax.axis_index("s")
    wid = sid * NC + cid

    zeros = jnp.zeros((L,), jnp.float32)
    ones = jnp.ones((L,), jnp.float32)

    pltpu.sync_copy(ei_hbm.at[0, pl.ds(wid * nchunk, nchunk)], src_v)
    pltpu.sync_copy(ei_hbm.at[1, pl.ds(wid * nchunk, nchunk)], dst_v)

    def zero_rows(i, _):
        r = i // (width // L)
        c = i % (width // L)
        rows[0][r, pl.ds(c * L, L)] = zeros
        return 0

    lax.fori_loop(0, chunk * (width // L), zero_rows, 0)
    for j in range(RPT // chunk):
        pltpu.sync_copy(rows[0], acc_sh.at[pl.ds(sid * RPT + j * chunk, chunk)])
    rem = RPT % chunk
    if rem:
        pltpu.sync_copy(rows[0].at[pl.ds(0, rem)],
                        acc_sh.at[pl.ds(sid * RPT + (RPT // chunk) * chunk, rem)])

    if with_deg:
        def zero_deg(i, _):
            r = i // (D // L)
            c = i % (D // L)
            deg_v[r, pl.ds(c * L, L)] = zeros
            return 0

        lax.fori_loop(0, DGR * (D // L), zero_deg, 0)
        riota = lax.iota(jnp.int32, L)
        for j in range(DGR // L):
            degidx_v[pl.ds(j * L, L)] = riota + (j * L)

        @pl.when(sid == 0)
        def _():
            nz = min(chunk, DGR)
            pltpu.sync_copy(rows[0].at[pl.ds(0, nz)], deg_sh.at[pl.ds(0, nz)])
            if DGR > nz:
                pltpu.sync_copy(rows[0].at[pl.ds(0, DGR - nz)],
                                deg_sh.at[pl.ds(nz, DGR - nz)])

    plsc.subcore_barrier()

    for b in range(nb):
        pltpu.async_copy(x_hbm.at[src_v.at[b]], rows[b], gsems[b])

    ngrp = chunk // L
    tail = chunk - ngrp * L
    tail_mask = lax.iota(jnp.int32, L) >= (L - tail)

    def deg_update(g):
        if with_deg:
            for k in range(ngrp):
                dv = dst_v[g, pl.ds(k * L, L)]
                plsc.addupdate_scatter(deg_v, [dv >> 7, dv & 127], ones)
            if tail:
                dv = dst_v[g, pl.ds(chunk - L, L)]
                plsc.addupdate_scatter(deg_v, [dv >> 7, dv & 127], ones,
                                       mask=tail_mask)

    nouter = nchunk // nb

    def outer(o, _):
        for b in range(nb):
            g = o * nb + b
            deg_update(g)
            pltpu.make_async_copy(x_hbm.at[src_v.at[g]], rows[b],
                                  gsems[b]).wait()
            pltpu.sync_copy(rows[b], acc_sh.at[dst_v.at[g]], add=True)
            pltpu.async_copy(x_hbm.at[src_v.at[g + nb]], rows[b], gsems[b])
        return 0

    lax.fori_loop(0, nouter - 1, outer, 0)
    for b in range(nb):
        g = (nouter - 1) * nb + b
        deg_update(g)
        pltpu.make_async_copy(x_hbm.at[src_v.at[g]], rows[b], gsems[b]).wait()
        pltpu.sync_copy(rows[b], acc_sh.at[dst_v.at[g]], add=True)
    for t in range(nchunk - nouter * nb):
        g = nouter * nb + t
        deg_update(g)
        pltpu.async_copy(x_hbm.at[src_v.at[g]], rows[0], gsems[0]).wait()
        pltpu.sync_copy(rows[0], acc_sh.at[dst_v.at[g]], add=True)

    if with_deg:
        pltpu.sync_copy(deg_v, deg_sh.at[degidx_v], add=True)

    plsc.subcore_barrier()
    pltpu.sync_copy(acc_sh.at[pl.ds(sid * RPT, RPT)], out_hbm.at[cid, sid])
    if with_deg:
        @pl.when(sid == 0)
        def _():
            pltpu.sync_copy(deg_sh, deg_hbm.at[cid])


def _seg_sum_call(x, ei, width, chunk, nb, with_deg):
    nchunk = EPW // chunk
    out_type = [jax.ShapeDtypeStruct((NC, NS, RPT, width), jnp.float32)]
    if with_deg:
        out_type.append(jax.ShapeDtypeStruct((NC, DGR, D), jnp.float32))
    scratch = [
        pltpu.VMEM((nchunk, chunk), jnp.int32),
        pltpu.VMEM((nchunk, chunk), jnp.int32),
    ]
    scratch += [pltpu.VMEM((chunk, width), jnp.float32) for _ in range(nb)]
    scratch += [pltpu.SemaphoreType.DMA for _ in range(nb)]
    if with_deg:
        scratch.append(pltpu.VMEM((DGR, D), jnp.float32))
        scratch.append(pltpu.VMEM((DGR,), jnp.int32))
    scratch.append(pltpu.VMEM_SHARED((NPAD, width), jnp.float32))
    if with_deg:
        scratch.append(pltpu.VMEM_SHARED((DGR, D), jnp.float32))
    fn = pl.kernel(
        functools.partial(_seg_sum_body, width, chunk, nb, with_deg),
        out_type=out_type,
        mesh=_mesh,
        scratch_types=scratch,
        compiler_params=pltpu.CompilerParams(needs_layout_passes=False,
                                             use_tc_tiling_on_sc=False),
    )
    return fn(x, ei)


def _layer1_tc(x, aggx, degt, W_self1, W_neigh1, b1, Wn2p, Ws2p):
    R = 2000

    def body(x_b, aggx_b, degt_b, ws1, wn1, b1_b, wn2, ws2, y2_b, hs2_b, inv_b):
        deg = degt_b[:, 0] + degt_b[:, 1]
        inv = 1.0 / jnp.clip(deg, 1.0, None)
        mean = (aggx_b[0] + aggx_b[1]) * inv[:, None]
        xb = x_b[...].astype(jnp.bfloat16)
        h1 = jnp.dot(xb, ws1[...], preferred_element_type=jnp.float32)
        h1 += jnp.dot(mean.astype(jnp.bfloat16), wn1[...],
                      preferred_element_type=jnp.float32)
        h1 = jnp.maximum(h1 + b1_b[...], 0.0).astype(jnp.bfloat16)
        y2_b[...] = jnp.dot(h1, wn2[...], preferred_element_type=jnp.float32)
        hs2_b[...] = jnp.dot(h1, ws2[...], preferred_element_type=jnp.float32)
        inv_b[...] = jnp.broadcast_to(inv[:, None], (R, 8))

    grid = (N // R,)
    return pl.pallas_call(
        body,
        grid=grid,
        in_specs=[
            pl.BlockSpec((R, D), lambda i: (i, 0)),
            pl.BlockSpec((NC, R, D), lambda i: (0, i, 0)),
            pl.BlockSpec((R, 2), lambda i: (i, 0)),
            pl.BlockSpec((D, H), lambda i: (0, 0)),
            pl.BlockSpec((D, H), lambda i: (0, 0)),
            pl.BlockSpec((1, H), lambda i: (0, 0)),
            pl.BlockSpec((H, CP), lambda i: (0, 0)),
            pl.BlockSpec((H, CP), lambda i: (0, 0)),
        ],
        out_specs=[
            pl.BlockSpec((R, CP), lambda i: (i, 0)),
            pl.BlockSpec((R, CP), lambda i: (i, 0)),
            pl.BlockSpec((R, 8), lambda i: (i, 0)),
        ],
        out_shape=[
            jax.ShapeDtypeStruct((N, CP), jnp.float32),
            jax.ShapeDtypeStruct((N, CP), jnp.float32),
            jax.ShapeDtypeStruct((N, 8), jnp.float32),
        ],
    )(x, aggx, degt, W_self1, W_neigh1, b1, Wn2p, Ws2p)


def _layer2_tc(hs2, agg2, invd, b2p):
    R = 2000

    def body(hs2_b, agg2_b, inv_b, b2_b, out_b):
        inv = inv_b[:, 0]
        full = hs2_b[...] + (agg2_b[0] + agg2_b[1]) * inv[:, None] + b2_b[...]
        out_b[...] = full[:, :C]

    grid = (N // R,)
    return pl.pallas_call(
        body,
        grid=grid,
        in_specs=[
            pl.BlockSpec((R, CP), lambda i: (i, 0)),
            pl.BlockSpec((NC, R, CP), lambda i: (0, i, 0)),
            pl.BlockSpec((R, 8), lambda i: (i, 0)),
            pl.BlockSpec((1, CP), lambda i: (0, 0)),
        ],
        out_specs=pl.BlockSpec((R, C), lambda i: (i, 0)),
        out_shape=jax.ShapeDtypeStruct((N, C), jnp.float32),
    )(hs2, agg2, invd, b2p)


def kernel(x, edge_index, W_self1, W_neigh1, b1, W_self2, W_neigh2, b2):
    Wn2p = jnp.pad(W_neigh2, ((0, 0), (0, CP - C))).astype(jnp.bfloat16)
    Ws2p = jnp.pad(W_self2, ((0, 0), (0, CP - C))).astype(jnp.bfloat16)
    Ws1h = W_self1.astype(jnp.bfloat16)
    Wn1h = W_neigh1.astype(jnp.bfloat16)
    b1r = b1.reshape(1, H)
    b2p = jnp.pad(b2, (0, CP - C)).reshape(1, CP)

    c1, nb1 = 50, 2
    c2, nb2 = 125, 4
    aggx, degp = _seg_sum_call(x, edge_index.reshape(2, E // c1, c1),
                               D, c1, nb1, True)
    aggx = aggx.reshape(NC, NPAD, D)
    degt = degp.reshape(NC, NPAD).T
    y2p, hs2, invd = _layer1_tc(x, aggx, degt, Ws1h, Wn1h, b1r, Wn2p, Ws2p)
    agg2, = _seg_sum_call(y2p, edge_index.reshape(2, E // c2, c2),
                          CP, c2, nb2, False)
    agg2 = agg2.reshape(NC, NPAD, CP)
    return _layer2_tc(hs2, agg2, invd, b2p)


# speedup vs baseline: 1.1804x; 1.1804x over previous
import functools

import jax
import jax.numpy as jnp
from jax import lax
from jax.experimental import pallas as pl
from jax.experimental.pallas import tpu as pltpu
from jax.experimental.pallas import tpu_sc as plsc

N = 10000
E = 320000
D = 128
H = 128
C = 40
CP = 48

NC = 2
NS = 16
L = 16
NW = NC * NS
EPW = E // NW
RPT = N // NS
DGR = 80

_mesh = plsc.VectorSubcoreMesh(core_axis_name="c", subcore_axis_name="s")


def _seg_sum_body(width, chunk, nb, with_deg, x_hbm, ei_hbm, *rest):
    nchunk = EPW // chunk
    if with_deg:
        out_hbm, deg_hbm = rest[:2]
        rest = rest[2:]
    else:
        out_hbm = rest[0]
        deg_hbm = None
        rest = rest[1:]
    src_f, dstc_v = rest[:2]
    rows = list(rest[2:2 + nb])
    gsems = list(rest[2 + nb:2 + 2 * nb])
    isems = list(rest[2 + 2 * nb:2 + 3 * nb])
    rest = rest[2 + 3 * nb:]
    if with_deg:
        deg_v, degidx_v, acc_sh, deg_sh = rest[:4]
    else:
        deg_v = degidx_v = deg_sh = None
        acc_sh = rest[0]

    cid = lax.axis_index("c")
    sid = lax.axis_index("s")
    wid = sid * NC + cid
    ebase = wid * EPW

    zeros = jnp.zeros((L,), jnp.float32)
    ones = jnp.ones((L,), jnp.float32)

    pltpu.sync_copy(ei_hbm.at[0, pl.ds(ebase, EPW)], src_f)

    def zero_rows(i, _):
        r = i // (width // L)
        c = i % (width // L)
        rows[0][r, pl.ds(c * L, L)] = zeros
        return 0

    lax.fori_loop(0, chunk * (width // L), zero_rows, 0)
    for j in range(RPT // chunk):
        pltpu.sync_copy(rows[0], acc_sh.at[pl.ds(sid * RPT + j * chunk, chunk)])
    rem = RPT % chunk
    if rem:
        pltpu.sync_copy(rows[0].at[pl.ds(0, rem)],
                        acc_sh.at[pl.ds(sid * RPT + (RPT // chunk) * chunk, rem)])

    if with_deg:
        def zero_deg(i, _):
            r = i // (D // L)
            c = i % (D // L)
            deg_v[r, pl.ds(c * L, L)] = zeros
            return 0

        lax.fori_loop(0, DGR * (D // L), zero_deg, 0)
        riota = lax.iota(jnp.int32, L)
        for j in range(DGR // L):
            degidx_v[pl.ds(j * L, L)] = riota + (j * L)

        @pl.when(sid == 0)
        def _():
            nz = min(chunk, DGR)
            pltpu.sync_copy(rows[0].at[pl.ds(0, nz)], deg_sh.at[pl.ds(0, nz)])
            if DGR > nz:
                pltpu.sync_copy(rows[0].at[pl.ds(0, DGR - nz)],
                                deg_sh.at[pl.ds(nz, DGR - nz)])

    plsc.subcore_barrier()

    def fire(g, b):
        pltpu.async_copy(ei_hbm.at[1, pl.ds(ebase + g * chunk, chunk)],
                         dstc_v.at[b], isems[b])
        pltpu.async_copy(x_hbm.at[src_f.at[pl.ds(g * chunk, chunk)]],
                         rows[b], gsems[b])

    def drain(g, b):
        pltpu.make_async_copy(ei_hbm.at[1, pl.ds(ebase + g * chunk, chunk)],
                              dstc_v.at[b], isems[b]).wait()
        if with_deg:
            for k in range(chunk // L):
                dv = dstc_v[b, pl.ds(k * L, L)]
                plsc.addupdate_scatter(deg_v, [dv >> 7, dv & 127], ones)
        pltpu.make_async_copy(x_hbm.at[src_f.at[pl.ds(g * chunk, chunk)]],
                              rows[b], gsems[b]).wait()
        pltpu.sync_copy(rows[b], acc_sh.at[dstc_v.at[b]], add=True)

    for b in range(nb):
        fire(b, b)

    nouter = nchunk // nb

    def outer(o, _):
        for b in range(nb):
            g = o * nb + b
            drain(g, b)
            fire(g + nb, b)
        return 0

    lax.fori_loop(0, nouter - 1, outer, 0)
    for b in range(nb):
        drain((nouter - 1) * nb + b, b)
    for t in range(nchunk - nouter * nb):
        g = nouter * nb + t
        fire(g, 0)
        drain(g, 0)

    if with_deg:
        pltpu.sync_copy(deg_v, deg_sh.at[degidx_v], add=True)

    plsc.subcore_barrier()
    pltpu.sync_copy(acc_sh.at[pl.ds(sid * RPT, RPT)],
                    out_hbm.at[cid, pl.ds(sid * RPT, RPT)])
    if with_deg:
        @pl.when(sid == 0)
        def _():
            pltpu.sync_copy(deg_sh, deg_hbm.at[cid])


def _seg_sum_call(x, ei, width, chunk, nb, with_deg):
    out_type = [jax.ShapeDtypeStruct((NC, N, width), jnp.float32)]
    if with_deg:
        out_type.append(jax.ShapeDtypeStruct((NC, DGR, D), jnp.float32))
    scratch = [
        pltpu.VMEM((EPW,), jnp.int32),
        pltpu.VMEM((nb, chunk), jnp.int32),
    ]
    scratch += [pltpu.VMEM((chunk, width), jnp.float32) for _ in range(nb)]
    scratch += [pltpu.SemaphoreType.DMA for _ in range(2 * nb)]
    if with_deg:
        scratch.append(pltpu.VMEM((DGR, D), jnp.float32))
        scratch.append(pltpu.VMEM((DGR,), jnp.int32))
    scratch.append(pltpu.VMEM_SHARED((N, width), jnp.float32))
    if with_deg:
        scratch.append(pltpu.VMEM_SHARED((DGR, D), jnp.float32))
    fn = pl.kernel(
        functools.partial(_seg_sum_body, width, chunk, nb, with_deg),
        out_type=out_type,
        mesh=_mesh,
        scratch_types=scratch,
        compiler_params=pltpu.CompilerParams(needs_layout_passes=False,
                                             use_tc_tiling_on_sc=False),
    )
    return fn(x, ei)


def _layer1_tc(x, aggx, degt, W_self1, W_neigh1, b1, Wn2p, Ws2p):
    R = 2000

    def body(x_b, aggx_b, degt_b, ws1, wn1, b1_b, wn2, ws2, y2_b, hs2_b, inv_b):
        deg = degt_b[:, 0] + degt_b[:, 1]
        inv = 1.0 / jnp.clip(deg, 1.0, None)
        mean = (aggx_b[0] + aggx_b[1]) * inv[:, None]
        xb = x_b[...].astype(jnp.bfloat16)
        h1 = jnp.dot(xb, ws1[...], preferred_element_type=jnp.float32)
        h1 += jnp.dot(mean.astype(jnp.bfloat16), wn1[...],
                      preferred_element_type=jnp.float32)
        h1 = jnp.maximum(h1 + b1_b[...], 0.0).astype(jnp.bfloat16)
        y2_b[...] = jnp.dot(h1, wn2[...], preferred_element_type=jnp.float32)
        hs2_b[...] = jnp.dot(h1, ws2[...], preferred_element_type=jnp.float32)
        inv_b[...] = jnp.broadcast_to(inv[:, None], (R, 8))

    grid = (N // R,)
    return pl.pallas_call(
        body,
        grid=grid,
        in_specs=[
            pl.BlockSpec((R, D), lambda i: (i, 0)),
            pl.BlockSpec((NC, R, D), lambda i: (0, i, 0)),
            pl.BlockSpec((R, 2), lambda i: (i, 0)),
            pl.BlockSpec((D, H), lambda i: (0, 0)),
            pl.BlockSpec((D, H), lambda i: (0, 0)),
            pl.BlockSpec((1, H), lambda i: (0, 0)),
            pl.BlockSpec((H, CP), lambda i: (0, 0)),
            pl.BlockSpec((H, CP), lambda i: (0, 0)),
        ],
        out_specs=[
            pl.BlockSpec((R, CP), lambda i: (i, 0)),
            pl.BlockSpec((R, CP), lambda i: (i, 0)),
            pl.BlockSpec((R, 8), lambda i: (i, 0)),
        ],
        out_shape=[
            jax.ShapeDtypeStruct((N, CP), jnp.float32),
            jax.ShapeDtypeStruct((N, CP), jnp.float32),
            jax.ShapeDtypeStruct((N, 8), jnp.float32),
        ],
    )(x, aggx, degt, W_self1, W_neigh1, b1, Wn2p, Ws2p)


def _layer2_tc(hs2, agg2, invd, b2p):
    R = 2000

    def body(hs2_b, agg2_b, inv_b, b2_b, out_b):
        inv = inv_b[:, 0]
        full = hs2_b[...] + (agg2_b[0] + agg2_b[1]) * inv[:, None] + b2_b[...]
        out_b[...] = full[:, :C]

    grid = (N // R,)
    return pl.pallas_call(
        body,
        grid=grid,
        in_specs=[
            pl.BlockSpec((R, CP), lambda i: (i, 0)),
            pl.BlockSpec((NC, R, CP), lambda i: (0, i, 0)),
            pl.BlockSpec((R, 8), lambda i: (i, 0)),
            pl.BlockSpec((1, CP), lambda i: (0, 0)),
        ],
        out_specs=pl.BlockSpec((R, C), lambda i: (i, 0)),
        out_shape=jax.ShapeDtypeStruct((N, C), jnp.float32),
    )(hs2, agg2, invd, b2p)


def kernel(x, edge_index, W_self1, W_neigh1, b1, W_self2, W_neigh2, b2):
    Wn2p = jnp.pad(W_neigh2, ((0, 0), (0, CP - C))).astype(jnp.bfloat16)
    Ws2p = jnp.pad(W_self2, ((0, 0), (0, CP - C))).astype(jnp.bfloat16)
    Ws1h = W_self1.astype(jnp.bfloat16)
    Wn1h = W_neigh1.astype(jnp.bfloat16)
    b1r = b1.reshape(1, H)
    b2p = jnp.pad(b2, (0, CP - C)).reshape(1, CP)

    aggx, degp = _seg_sum_call(x, edge_index, D, 80, 2, True)
    degt = degp.reshape(NC, DGR * D)[:, :N].T
    y2p, hs2, invd = _layer1_tc(x, aggx, degt, Ws1h, Wn1h, b1r, Wn2p, Ws2p)
    agg2, = _seg_sum_call(y2p, edge_index, CP, 80, 4, False)
    return _layer2_tc(hs2, agg2, invd, b2p)
